# baseline (speedup 1.0000x reference)
import functools

import jax
import jax.numpy as jnp
from jax import lax
from jax.experimental import pallas as pl
from jax.experimental.pallas import tpu as pltpu
from jax.experimental.pallas import tpu_sc as plsc

N_NODES = 10000
N_EDGES = 320000
D = 128

NC = 2
NS = 16
NW = NC * NS

CH = 50
G = 25
NBUF1 = 4
NBUF2 = 3
CH2 = 100
G2 = 10
E_PER_W = N_EDGES // NW
NG = E_PER_W // (G * CH)
NG2 = E_PER_W // (G2 * CH2)
ROWS_PER_TILE = N_NODES // NS
ZCH = 125
CBLK = 1000
NPAD = 10240

_MESH = plsc.VectorSubcoreMesh(core_axis_name="c", subcore_axis_name="s")


def _zero_agg(s, zeros_hbm, agg_sh):
    for k in range(ROWS_PER_TILE // ZCH):
        sl = pl.ds(s * ROWS_PER_TILE + k * ZCH, ZCH)
        pltpu.sync_copy(zeros_hbm, agg_sh.at[sl])


def _edge_pipeline(x_hbm, srcs_hbm, dsts_hbm, wid, src_g, dst_g, rows, gsems,
                   ssems, agg_sh, ng, count_chunk=None):
    NBUF = len(rows)
    G = src_g.shape[0]
    NG = ng

    def fire_gather(k, b):
        pltpu.async_copy(x_hbm.at[src_g.at[k]], rows[b], gsems[b])

    def wait_gather(b):
        pltpu.make_async_copy(x_hbm.at[src_g.at[0]], rows[b], gsems[b]).wait()

    def fire_scatter(k, b):
        pltpu.async_copy(rows[b], agg_sh.at[dst_g.at[k]], ssems[b], add=True)

    def wait_scatter(b):
        pltpu.make_async_copy(rows[b], agg_sh.at[dst_g.at[0]], ssems[b]).wait()

    def group(g, is_first):
        for j in range(NBUF - 1):
            if not is_first:
                wait_scatter(j)
            fire_gather(j, j)
        for k in range(G):
            b = k % NBUF
            ahead = k + NBUF - 1
            if ahead < G:
                ab = ahead % NBUF
                if not (is_first and ahead == NBUF - 1):
                    wait_scatter(ab)
                fire_gather(ahead, ab)
            wait_gather(b)
            fire_scatter(k, b)
            if count_chunk is not None:
                count_chunk(k)
        if is_first:
            pltpu.sync_copy(srcs_hbm.at[wid, 1], src_g)
            pltpu.sync_copy(dsts_hbm.at[wid, 1], dst_g)
        else:
            @pl.when(g < NG - 1)
            def _():
                pltpu.sync_copy(srcs_hbm.at[wid, g + 1], src_g)
                pltpu.sync_copy(dsts_hbm.at[wid, g + 1], dst_g)

    pltpu.sync_copy(srcs_hbm.at[wid, 0], src_g)
    pltpu.sync_copy(dsts_hbm.at[wid, 0], dst_g)
    group(0, True)

    def gbody(g, carry):
        group(g, False)
        return carry

    lax.fori_loop(1, NG, gbody, 0)
    for b in range(NBUF):
        wait_scatter(b)


def _sc_agg_cnt_body(x_hbm, srcs_hbm, dsts_hbm, zeros_hbm,
                     out_agg, out_cnt, src_g, dst_g, *bufs):
    rows = bufs[:NBUF1]
    cnt_v, agg_sh = bufs[NBUF1], bufs[NBUF1 + 1]
    gsems = bufs[NBUF1 + 2:2 * NBUF1 + 2]
    ssems = bufs[2 * NBUF1 + 2:]
    c = lax.axis_index("c")
    s = lax.axis_index("s")
    wid = c * NS + s
    _zero_agg(s, zeros_hbm, agg_sh)
    zeros16 = jnp.zeros((16,), jnp.float32)

    def zstep(v, carry):
        cnt_v[pl.ds(v * 16, 16)] = zeros16
        return carry

    lax.fori_loop(0, NPAD // 16, zstep, 0)
    ones16 = jnp.ones((16,), jnp.float32)
    tail_mask = lax.iota(jnp.int32, 16) >= (16 - (CH - 3 * 16))

    def count_chunk(k):
        for off in range(0, 48, 16):
            idx = dst_g[k, pl.ds(off, 16)]
            plsc.addupdate_scatter(cnt_v, [idx], ones16)
        idx = dst_g[k, pl.ds(CH - 16, 16)]
        plsc.addupdate_scatter(cnt_v, [idx], ones16, mask=tail_mask)

    plsc.subcore_barrier()
    _edge_pipeline(x_hbm, srcs_hbm, dsts_hbm, wid, src_g, dst_g,
                   rows, gsems, ssems, agg_sh, NG, count_chunk=count_chunk)
    plsc.subcore_barrier()
    pltpu.sync_copy(cnt_v, out_cnt.at[wid])
    @pl.when(s == 0)
    def _():
        pltpu.sync_copy(agg_sh, out_agg.at[c])


def _sc_agg_body(x_hbm, srcs_hbm, dsts_hbm, zeros_hbm, out_agg,
                 src_g, dst_g, *bufs):
    rows = bufs[:NBUF2]
    agg_sh = bufs[NBUF2]
    gsems = bufs[NBUF2 + 1:2 * NBUF2 + 1]
    ssems = bufs[2 * NBUF2 + 1:]
    c = lax.axis_index("c")
    s = lax.axis_index("s")
    wid = c * NS + s
    _zero_agg(s, zeros_hbm, agg_sh)
    plsc.subcore_barrier()
    _edge_pipeline(x_hbm, srcs_hbm, dsts_hbm, wid, src_g, dst_g,
                   rows, gsems, ssems, agg_sh, NG2)
    plsc.subcore_barrier()
    @pl.when(s == 0)
    def _():
        pltpu.sync_copy(agg_sh, out_agg.at[c])


_sc_agg_cnt = pl.kernel(
    _sc_agg_cnt_body,
    out_type=(jax.ShapeDtypeStruct((NC, N_NODES, D), jnp.float32),
              jax.ShapeDtypeStruct((NW, NPAD), jnp.float32)),
    mesh=_MESH,
    compiler_params=pltpu.CompilerParams(needs_layout_passes=False),
    scratch_types=(
        [pltpu.VMEM((G, CH), jnp.int32)] * 2
        + [pltpu.VMEM((CH, D), jnp.float32)] * NBUF1
        + [pltpu.VMEM((NPAD,), jnp.float32),
           pltpu.VMEM_SHARED((N_NODES, D), jnp.float32)]
        + [pltpu.SemaphoreType.DMA] * (2 * NBUF1)
    ),
)

_sc_agg = pl.kernel(
    _sc_agg_body,
    out_type=jax.ShapeDtypeStruct((NC, N_NODES, D), jnp.float32),
    mesh=_MESH,
    compiler_params=pltpu.CompilerParams(needs_layout_passes=False),
    scratch_types=(
        [pltpu.VMEM((G2, CH2), jnp.int32)] * 2
        + [pltpu.VMEM((CH2, D), jnp.float32)] * NBUF2
        + [pltpu.VMEM_SHARED((N_NODES, D), jnp.float32)]
        + [pltpu.SemaphoreType.DMA] * (2 * NBUF2)
    ),
)


def _lin_body(relu, agg_ref, cnt_ref, x_ref, wl_ref, wr_ref, b_ref, o_ref):
    aggsum = agg_ref[0] + agg_ref[1]
    cnt = jnp.sum(cnt_ref[...], axis=1)[:, None]
    mean = aggsum / jnp.maximum(cnt, 1.0)
    y = jnp.dot(mean, wl_ref[...], preferred_element_type=jnp.float32)
    y = y + jnp.dot(x_ref[...], wr_ref[...], preferred_element_type=jnp.float32)
    y = y + b_ref[...][None, :]
    if relu:
        y = jnp.maximum(y, 0.0)
    o_ref[...] = y


def _linear(agg, cnt, x, wl, wr, b, relu):
    blk = CBLK
    return pl.pallas_call(
        functools.partial(_lin_body, relu),
        grid=(N_NODES // blk,),
        in_specs=[
            pl.BlockSpec((NC, blk, D), lambda i: (0, i, 0)),
            pl.BlockSpec((blk, NW), lambda i: (i, 0)),
            pl.BlockSpec((blk, D), lambda i: (i, 0)),
            pl.BlockSpec((D, D), lambda i: (0, 0)),
            pl.BlockSpec((D, D), lambda i: (0, 0)),
            pl.BlockSpec((D,), lambda i: (0,)),
        ],
        out_specs=pl.BlockSpec((blk, D), lambda i: (i, 0)),
        out_shape=jax.ShapeDtypeStruct((N_NODES, D), jnp.float32),
    )(agg, cnt, x, wl, wr, b)


def kernel(x, edge_index, W1_l, W1_r, b1, W2_l, W2_r, b2):
    ei = edge_index.astype(jnp.int32)
    srcs = ei[0].reshape(NW, NG, G, CH)
    dsts = ei[1].reshape(NW, NG, G, CH)
    zeros = jnp.zeros((ZCH, D), jnp.float32)
    agg1, cnt = _sc_agg_cnt(x, srcs, dsts, zeros)
    cnt_t = cnt[:, :N_NODES].T
    h = _linear(agg1, cnt_t, x, W1_l, W1_r, b1, relu=True)
    srcs2 = ei[0].reshape(NW, NG2, G2, CH2)
    dsts2 = ei[1].reshape(NW, NG2, G2, CH2)
    agg2 = _sc_agg(h, srcs2, dsts2, zeros)
    return _linear(agg2, cnt_t, h, W2_l, W2_r, b2, relu=False)


# speedup vs baseline: 1.0278x; 1.0278x over previous
import functools

import jax
import jax.numpy as jnp
from jax import lax
from jax.experimental import pallas as pl
from jax.experimental.pallas import tpu as pltpu
from jax.experimental.pallas import tpu_sc as plsc

N_NODES = 10000
N_EDGES = 320000
D = 128

NC = 2
NS = 16
NW = NC * NS

CH = 50
G = 25
NBUF1 = 4
NBUF2 = 5
CH2 = 50
G2 = 25
E_PER_W = N_EDGES // NW
NG = E_PER_W // (G * CH)
NG2 = E_PER_W // (G2 * CH2)
ROWS_PER_TILE = N_NODES // NS
ZCH = 125
CBLK = 1000
NPAD = 10240

_MESH = plsc.VectorSubcoreMesh(core_axis_name="c", subcore_axis_name="s")


def _zero_agg(s, zeros_hbm, agg_sh):
    for k in range(ROWS_PER_TILE // ZCH):
        sl = pl.ds(s * ROWS_PER_TILE + k * ZCH, ZCH)
        pltpu.sync_copy(zeros_hbm, agg_sh.at[sl])


def _edge_pipeline(x_hbm, srcs_hbm, dsts_hbm, wid, src_g, dst_g, rows, gsems,
                   ssems, agg_sh, ng, count_chunk=None):
    NBUF = len(rows)
    G = src_g.shape[0]
    NG = ng

    def fire_gather(k, b):
        pltpu.async_copy(x_hbm.at[src_g.at[k]], rows[b], gsems[b])

    def wait_gather(b):
        pltpu.make_async_copy(x_hbm.at[src_g.at[0]], rows[b], gsems[b]).wait()

    def fire_scatter(k, b):
        pltpu.async_copy(rows[b], agg_sh.at[dst_g.at[k]], ssems[b], add=True)

    def wait_scatter(b):
        pltpu.make_async_copy(rows[b], agg_sh.at[dst_g.at[0]], ssems[b]).wait()

    def group(g, is_first):
        for j in range(NBUF - 1):
            if not is_first:
                wait_scatter(j)
            fire_gather(j, j)
        for k in range(G):
            b = k % NBUF
            ahead = k + NBUF - 1
            if ahead < G:
                ab = ahead % NBUF
                if not (is_first and ahead == NBUF - 1):
                    wait_scatter(ab)
                fire_gather(ahead, ab)
            wait_gather(b)
            fire_scatter(k, b)
            if count_chunk is not None:
                count_chunk(k)
        if is_first:
            pltpu.sync_copy(srcs_hbm.at[wid, 1], src_g)
            pltpu.sync_copy(dsts_hbm.at[wid, 1], dst_g)
        else:
            @pl.when(g < NG - 1)
            def _():
                pltpu.sync_copy(srcs_hbm.at[wid, g + 1], src_g)
                pltpu.sync_copy(dsts_hbm.at[wid, g + 1], dst_g)

    pltpu.sync_copy(srcs_hbm.at[wid, 0], src_g)
    pltpu.sync_copy(dsts_hbm.at[wid, 0], dst_g)
    group(0, True)

    def gbody(g, carry):
        group(g, False)
        return carry

    lax.fori_loop(1, NG, gbody, 0)
    for b in range(NBUF):
        wait_scatter(b)


def _sc_agg_cnt_body(x_hbm, srcs_hbm, dsts_hbm, zeros_hbm,
                     out_agg, out_cnt, src_g, dst_g, *bufs):
    rows = bufs[:NBUF1]
    cnt_v, agg_sh = bufs[NBUF1], bufs[NBUF1 + 1]
    gsems = bufs[NBUF1 + 2:2 * NBUF1 + 2]
    ssems = bufs[2 * NBUF1 + 2:]
    c = lax.axis_index("c")
    s = lax.axis_index("s")
    wid = c * NS + s
    _zero_agg(s, zeros_hbm, agg_sh)
    zeros16 = jnp.zeros((16,), jnp.float32)

    def zstep(v, carry):
        cnt_v[pl.ds(v * 16, 16)] = zeros16
        return carry

    lax.fori_loop(0, NPAD // 16, zstep, 0)
    ones16 = jnp.ones((16,), jnp.float32)
    tail_mask = lax.iota(jnp.int32, 16) >= (16 - (CH - 3 * 16))

    def count_chunk(k):
        for off in range(0, 48, 16):
            idx = dst_g[k, pl.ds(off, 16)]
            plsc.addupdate_scatter(cnt_v, [idx], ones16)
        idx = dst_g[k, pl.ds(CH - 16, 16)]
        plsc.addupdate_scatter(cnt_v, [idx], ones16, mask=tail_mask)

    plsc.subcore_barrier()
    _edge_pipeline(x_hbm, srcs_hbm, dsts_hbm, wid, src_g, dst_g,
                   rows, gsems, ssems, agg_sh, NG, count_chunk=count_chunk)
    plsc.subcore_barrier()
    pltpu.sync_copy(cnt_v, out_cnt.at[wid])
    @pl.when(s == 0)
    def _():
        pltpu.sync_copy(agg_sh, out_agg.at[c])


def _sc_agg_body(x_hbm, srcs_hbm, dsts_hbm, zeros_hbm, out_agg,
                 src_g, dst_g, *bufs):
    rows = bufs[:NBUF2]
    agg_sh = bufs[NBUF2]
    gsems = bufs[NBUF2 + 1:2 * NBUF2 + 1]
    ssems = bufs[2 * NBUF2 + 1:]
    c = lax.axis_index("c")
    s = lax.axis_index("s")
    wid = c * NS + s
    _zero_agg(s, zeros_hbm, agg_sh)
    plsc.subcore_barrier()
    _edge_pipeline(x_hbm, srcs_hbm, dsts_hbm, wid, src_g, dst_g,
                   rows, gsems, ssems, agg_sh, NG2)
    plsc.subcore_barrier()
    @pl.when(s == 0)
    def _():
        pltpu.sync_copy(agg_sh, out_agg.at[c])


_sc_agg_cnt = pl.kernel(
    _sc_agg_cnt_body,
    out_type=(jax.ShapeDtypeStruct((NC, N_NODES, D), jnp.float32),
              jax.ShapeDtypeStruct((NW, NPAD), jnp.float32)),
    mesh=_MESH,
    compiler_params=pltpu.CompilerParams(needs_layout_passes=False),
    scratch_types=(
        [pltpu.VMEM((G, CH), jnp.int32)] * 2
        + [pltpu.VMEM((CH, D), jnp.float32)] * NBUF1
        + [pltpu.VMEM((NPAD,), jnp.float32),
           pltpu.VMEM_SHARED((N_NODES, D), jnp.float32)]
        + [pltpu.SemaphoreType.DMA] * (2 * NBUF1)
    ),
)

_sc_agg = pl.kernel(
    _sc_agg_body,
    out_type=jax.ShapeDtypeStruct((NC, N_NODES, D), jnp.float32),
    mesh=_MESH,
    compiler_params=pltpu.CompilerParams(needs_layout_passes=False),
    scratch_types=(
        [pltpu.VMEM((G2, CH2), jnp.int32)] * 2
        + [pltpu.VMEM((CH2, D), jnp.float32)] * NBUF2
        + [pltpu.VMEM_SHARED((N_NODES, D), jnp.float32)]
        + [pltpu.SemaphoreType.DMA] * (2 * NBUF2)
    ),
)


def _lin_body(relu, agg_ref, cnt_ref, x_ref, wl_ref, wr_ref, b_ref, o_ref):
    aggsum = agg_ref[0] + agg_ref[1]
    cnt = jnp.sum(cnt_ref[...], axis=1)[:, None]
    mean = aggsum / jnp.maximum(cnt, 1.0)
    y = jnp.dot(mean, wl_ref[...], preferred_element_type=jnp.float32)
    y = y + jnp.dot(x_ref[...], wr_ref[...], preferred_element_type=jnp.float32)
    y = y + b_ref[...][None, :]
    if relu:
        y = jnp.maximum(y, 0.0)
    o_ref[...] = y


def _linear(agg, cnt, x, wl, wr, b, relu):
    blk = CBLK
    return pl.pallas_call(
        functools.partial(_lin_body, relu),
        grid=(N_NODES // blk,),
        in_specs=[
            pl.BlockSpec((NC, blk, D), lambda i: (0, i, 0)),
            pl.BlockSpec((blk, NW), lambda i: (i, 0)),
            pl.BlockSpec((blk, D), lambda i: (i, 0)),
            pl.BlockSpec((D, D), lambda i: (0, 0)),
            pl.BlockSpec((D, D), lambda i: (0, 0)),
            pl.BlockSpec((D,), lambda i: (0,)),
        ],
        out_specs=pl.BlockSpec((blk, D), lambda i: (i, 0)),
        out_shape=jax.ShapeDtypeStruct((N_NODES, D), jnp.float32),
    )(agg, cnt, x, wl, wr, b)


def kernel(x, edge_index, W1_l, W1_r, b1, W2_l, W2_r, b2):
    ei = edge_index.astype(jnp.int32)
    srcs = ei[0].reshape(NW, NG, G, CH)
    dsts = ei[1].reshape(NW, NG, G, CH)
    zeros = jnp.zeros((ZCH, D), jnp.float32)
    agg1, cnt = _sc_agg_cnt(x, srcs, dsts, zeros)
    cnt_t = cnt[:, :N_NODES].T
    h = _linear(agg1, cnt_t, x, W1_l, W1_r, b1, relu=True)
    srcs2 = ei[0].reshape(NW, NG2, G2, CH2)
    dsts2 = ei[1].reshape(NW, NG2, G2, CH2)
    agg2 = _sc_agg(h, srcs2, dsts2, zeros)
    return _linear(agg2, cnt_t, h, W2_l, W2_r, b2, relu=False)
